# baseline (speedup 1.0000x reference)
import functools

import jax
import jax.numpy as jnp
from jax import lax
from jax.experimental import pallas as pl
from jax.experimental.pallas import tpu as pltpu
from jax.experimental.pallas import tpu_sc as plsc

_DIM = 128
_NC = 2
_NS = 16
_NW = _NC * _NS


def _make_sc_gather(n_idx, chunk):
    assert n_idx % (_NW * chunk) == 0 and chunk % 8 == 0
    per_w = n_idx // _NW
    nchunk = per_w // chunk
    assert nchunk % 2 == 0
    mesh = plsc.VectorSubcoreMesh(core_axis_name="c", subcore_axis_name="s")

    @functools.partial(
        pl.kernel,
        out_type=jax.ShapeDtypeStruct((n_idx, _DIM), jnp.float32),
        mesh=mesh,
        scratch_types=[
            pltpu.VMEM((chunk,), jnp.int32),
            pltpu.VMEM((chunk,), jnp.int32),
            pltpu.VMEM((chunk, _DIM), jnp.float32),
            pltpu.VMEM((chunk, _DIM), jnp.float32),
            pltpu.SemaphoreType.DMA,
            pltpu.SemaphoreType.DMA,
            pltpu.SemaphoreType.DMA,
            pltpu.SemaphoreType.DMA,
        ],
        compiler_params=pltpu.CompilerParams(needs_layout_passes=False),
    )
    def sc_kernel(idx_hbm, w_hbm, out_hbm, idx_a, idx_b, rows_a, rows_b,
                  sem_a, sem_b, osem_a, osem_b):
        wid = lax.axis_index("s") * _NC + lax.axis_index("c")
        base = wid * per_w
        idx_bufs = (idx_a, idx_b)
        row_bufs = (rows_a, rows_b)
        sems = (sem_a, sem_b)
        osems = (osem_a, osem_b)

        def fetch(c, b):
            pltpu.sync_copy(idx_hbm.at[pl.ds(base + c * chunk, chunk)],
                            idx_bufs[b])
            pltpu.async_copy(w_hbm.at[idx_bufs[b]], row_bufs[b], sems[b])

        fetch(0, 0)

        def pair_fn(i, _):
            for b in range(2):
                c = i * 2 + b
                nb = 1 - b

                @pl.when(c + 1 < nchunk)
                def _prefetch():
                    @pl.when(c >= 1)
                    def _drain():
                        pltpu.make_async_copy(
                            row_bufs[nb],
                            out_hbm.at[pl.ds(base, chunk)],
                            osems[nb]).wait()

                    fetch(c + 1, nb)

                pltpu.make_async_copy(
                    w_hbm.at[idx_bufs[b]], row_bufs[b], sems[b]).wait()
                pltpu.async_copy(row_bufs[b],
                                 out_hbm.at[pl.ds(base + c * chunk, chunk)],
                                 osems[b])
            return 0

        lax.fori_loop(0, nchunk // 2, pair_fn, 0)
        for b in range(2):
            pltpu.make_async_copy(row_bufs[b],
                                  out_hbm.at[pl.ds(base, chunk)],
                                  osems[b]).wait()

    return sc_kernel


def _tc_normalize_retile(rows, nb_out, l_out, slabs_per_blk):
    blk = slabs_per_blk * l_out

    def body(rows_ref, out_ref):
        for k in range(slabs_per_blk):
            v = rows_ref[pl.ds(k * l_out, l_out), :]
            ss = jnp.sum(v * v, axis=1, keepdims=True)
            out_ref[k] = v * lax.rsqrt(jnp.maximum(ss, 1e-24))

    return pl.pallas_call(
        body,
        grid=(nb_out // slabs_per_blk,),
        in_specs=[pl.BlockSpec((blk, _DIM), lambda i: (i, 0))],
        out_specs=pl.BlockSpec((slabs_per_blk, l_out, _DIM),
                               lambda i: (i, 0, 0)),
        out_shape=jax.ShapeDtypeStruct((nb_out, l_out, _DIM), jnp.float32),
    )(rows)


def kernel(x, weight):
    b, l = x.shape
    n_idx = b * l
    flat_idx = x.reshape(n_idx).astype(jnp.int32)
    rows = _make_sc_gather(n_idx, chunk=400)(flat_idx, weight)
    return _tc_normalize_retile(rows, b, l, slabs_per_blk=8)
